# baseline (speedup 1.0000x reference)
import functools

import jax
import jax.numpy as jnp
from jax.experimental import pallas as pl
from jax.experimental.pallas import tpu as pltpu


def _sigmoid(x):
    return 0.5 * jnp.tanh(0.5 * x) + 0.5


def _bilstm_mean_relu_kernel(xa_ref, xb_ref, wia_ref, whh_ref, b_ref,
                             out_ref, gx_ref, *, H, T, unroll):
    k = pl.program_id(1)
    n_k = pl.num_programs(1)
    B_blk = xa_ref.shape[0]
    _, _, G = gx_ref.shape
    GP = G // 4
    inv_T = 1.0 / T

    wia = wia_ref[...]
    bias = b_ref[...]

    lane = jax.lax.broadcasted_iota(jnp.int32, (1, G), 1) % GP
    bwd_mask = jnp.logical_and(lane >= H, lane < 2 * H)

    p1 = jnp.dot(xa_ref[...], wia, preferred_element_type=jnp.float32) + bias
    p2 = jnp.dot(xb_ref[...], wia, preferred_element_type=jnp.float32) + bias
    gx_ref[k] = jnp.where(bwd_mask, p2, p1)
    gx_ref[T - 1 - k] = jnp.where(bwd_mask, p1, p2)

    @pl.when(k == n_k - 1)
    def _recurrence():
        whh = whh_ref[...]

        def step(t, carry):
            h, c, acc = carry
            gates = gx_ref[t] + jnp.dot(h, whh,
                                        preferred_element_type=jnp.float32)
            i = _sigmoid(gates[:, 0 * GP:1 * GP])
            f = _sigmoid(gates[:, 1 * GP:2 * GP])
            g = jnp.tanh(gates[:, 2 * GP:3 * GP])
            o = _sigmoid(gates[:, 3 * GP:4 * GP])
            c = f * c + i * g
            h = o * jnp.tanh(c)
            return h, c, acc + h

        h = jnp.zeros((B_blk, GP), jnp.float32)
        c = jnp.zeros((B_blk, GP), jnp.float32)
        acc = jnp.zeros((B_blk, GP), jnp.float32)
        h, c, acc = jax.lax.fori_loop(0, T, step, (h, c, acc), unroll=unroll)

        out_ref[...] = jnp.maximum(acc * inv_T, 0.0)


def kernel(x, wia, whh, b):
    B, T, D = x.shape
    GP = whh.shape[0]
    G = wia.shape[1]
    H = GP // 2

    Bp = max(8, -(-B // 8) * 8)
    B_BLK = min(Bp, 512)
    Bp = -(-Bp // B_BLK) * B_BLK
    if Bp != B:
        x = jnp.pad(x, ((0, Bp - B), (0, 0), (0, 0)))

    assert T % 2 == 0
    body = functools.partial(_bilstm_mean_relu_kernel, H=H, T=T, unroll=4)

    xf = x.reshape(Bp, T * D)

    out = pl.pallas_call(
        body,
        out_shape=jax.ShapeDtypeStruct((Bp, GP), jnp.float32),
        grid=(Bp // B_BLK, T // 2),
        in_specs=[
            pl.BlockSpec((B_BLK, D), lambda i, k: (i, k)),
            pl.BlockSpec((B_BLK, D), lambda i, k: (i, T - 1 - k)),
            pl.BlockSpec(wia.shape, lambda i, k: (0, 0)),
            pl.BlockSpec(whh.shape, lambda i, k: (0, 0)),
            pl.BlockSpec(b.shape, lambda i, k: (0, 0)),
        ],
        out_specs=pl.BlockSpec((B_BLK, GP), lambda i, k: (i, 0)),
        scratch_shapes=[pltpu.VMEM((T, B_BLK, G), jnp.float32)],
        compiler_params=pltpu.CompilerParams(
            dimension_semantics=("parallel", "arbitrary"),
            vmem_limit_bytes=56 * 1024 * 1024,
        ),
    )(xf, xf, wia, whh, b)

    return out[:B, :GP]


# speedup vs baseline: 1.8505x; 1.8505x over previous
import functools

import jax
import jax.numpy as jnp
from jax.experimental import pallas as pl
from jax.experimental.pallas import tpu as pltpu


def _sigmoid(x):
    return 0.5 * jnp.tanh(0.5 * x) + 0.5


def _bilstm_mean_relu_kernel(x_hbm, wia_ref, whh_ref, b_ref, out_ref,
                             gx_ref, xbuf, sem, *, H, B_BLK, unroll):
    _, T, _ = x_hbm.shape
    _, _, G = gx_ref.shape
    GP = G // 4
    inv_T = 1.0 / T
    base = pl.program_id(0) * B_BLK
    n_pairs = T // 2

    def start_pair(k, slot):
        pltpu.make_async_copy(
            x_hbm.at[pl.ds(base, B_BLK), k, :],
            xbuf.at[slot, 0], sem.at[slot, 0]).start()
        pltpu.make_async_copy(
            x_hbm.at[pl.ds(base, B_BLK), T - 1 - k, :],
            xbuf.at[slot, 1], sem.at[slot, 1]).start()

    def wait_pair(k, slot):
        pltpu.make_async_copy(
            x_hbm.at[pl.ds(base, B_BLK), k, :],
            xbuf.at[slot, 0], sem.at[slot, 0]).wait()
        pltpu.make_async_copy(
            x_hbm.at[pl.ds(base, B_BLK), T - 1 - k, :],
            xbuf.at[slot, 1], sem.at[slot, 1]).wait()

    wia = wia_ref[...]
    bias = b_ref[...]

    lane = jax.lax.broadcasted_iota(jnp.int32, (1, G), 1) % GP
    bwd_mask = jnp.logical_and(lane >= H, lane < 2 * H)

    start_pair(0, 0)
    for k in range(n_pairs):
        if k + 1 < n_pairs:
            start_pair(k + 1, (k + 1) % 2)
        wait_pair(k, k % 2)
        p1 = jnp.dot(xbuf[k % 2, 0], wia,
                     preferred_element_type=jnp.float32) + bias
        p2 = jnp.dot(xbuf[k % 2, 1], wia,
                     preferred_element_type=jnp.float32) + bias
        gx_ref[k] = jnp.where(bwd_mask, p2, p1)
        gx_ref[T - 1 - k] = jnp.where(bwd_mask, p1, p2)

    whh = whh_ref[...]

    def step(t, carry):
        h, c, acc = carry
        gates = gx_ref[t] + jnp.dot(h, whh, preferred_element_type=jnp.float32)
        i = _sigmoid(gates[:, 0 * GP:1 * GP])
        f = _sigmoid(gates[:, 1 * GP:2 * GP])
        g = jnp.tanh(gates[:, 2 * GP:3 * GP])
        o = _sigmoid(gates[:, 3 * GP:4 * GP])
        c = f * c + i * g
        h = o * jnp.tanh(c)
        return h, c, acc + h

    h = jnp.zeros((B_BLK, GP), jnp.float32)
    c = jnp.zeros((B_BLK, GP), jnp.float32)
    acc = jnp.zeros((B_BLK, GP), jnp.float32)
    h, c, acc = jax.lax.fori_loop(0, T, step, (h, c, acc), unroll=unroll)

    out_ref[...] = jnp.maximum(acc * inv_T, 0.0)


def kernel(x, wia, whh, b):
    B, T, D = x.shape
    GP = whh.shape[0]
    G = wia.shape[1]
    H = GP // 2

    Bp = max(8, -(-B // 8) * 8)
    B_BLK = min(Bp, 512)
    Bp = -(-Bp // B_BLK) * B_BLK
    if Bp != B:
        x = jnp.pad(x, ((0, Bp - B), (0, 0), (0, 0)))

    assert T % 2 == 0
    body = functools.partial(_bilstm_mean_relu_kernel, H=H, B_BLK=B_BLK,
                             unroll=4)

    out = pl.pallas_call(
        body,
        out_shape=jax.ShapeDtypeStruct((Bp, GP), jnp.float32),
        grid=(Bp // B_BLK,),
        in_specs=[
            pl.BlockSpec(memory_space=pltpu.MemorySpace.HBM),
            pl.BlockSpec(wia.shape, lambda i: (0, 0)),
            pl.BlockSpec(whh.shape, lambda i: (0, 0)),
            pl.BlockSpec(b.shape, lambda i: (0, 0)),
        ],
        out_specs=pl.BlockSpec((B_BLK, GP), lambda i: (i, 0)),
        scratch_shapes=[
            pltpu.VMEM((T, B_BLK, G), jnp.float32),
            pltpu.VMEM((2, 2, B_BLK, D), jnp.float32),
            pltpu.SemaphoreType.DMA((2, 2)),
        ],
        compiler_params=pltpu.CompilerParams(
            dimension_semantics=("parallel",),
            vmem_limit_bytes=56 * 1024 * 1024,
        ),
    )(x, wia, whh, b)

    return out[:B, :GP]


# speedup vs baseline: 2.1634x; 1.1691x over previous
import functools

import jax
import jax.numpy as jnp
from jax.experimental import pallas as pl
from jax.experimental.pallas import tpu as pltpu

_NSLOTS = 4


def _bilstm_mean_relu_kernel(x_hbm, wia_ref, whh_ref, b_ref, out_ref,
                             gx_ref, xbuf, sem, *, H, B_BLK, unroll):
    _, T, _ = x_hbm.shape
    _, _, G = gx_ref.shape
    GP = G // 4
    inv_T = 1.0 / T
    base = pl.program_id(0) * B_BLK
    n_pairs = T // 2

    def pair_copies(k, slot):
        return (pltpu.make_async_copy(x_hbm.at[pl.ds(base, B_BLK), k, :],
                                      xbuf.at[slot, 0], sem.at[slot, 0]),
                pltpu.make_async_copy(x_hbm.at[pl.ds(base, B_BLK), T - 1 - k, :],
                                      xbuf.at[slot, 1], sem.at[slot, 1]))

    wia = wia_ref[...]
    bias = b_ref[...]

    lane = jax.lax.broadcasted_iota(jnp.int32, (1, G), 1) % GP
    bwd_mask = jnp.logical_and(lane >= H, lane < 2 * H)

    whh = whh_ref[...]

    def step(gates, h, c, acc):
        ti = jnp.tanh(gates[:, 0 * GP:1 * GP])
        tf = jnp.tanh(gates[:, 1 * GP:2 * GP])
        g = jnp.tanh(gates[:, 2 * GP:3 * GP])
        to = jnp.tanh(gates[:, 3 * GP:4 * GP])
        c = 0.5 * (tf * c + c + ti * g + g)
        tc = jnp.tanh(c)
        h = 0.5 * (to * tc + tc)
        return h, c, acc + h

    h = jnp.zeros((B_BLK, GP), jnp.float32)
    c = jnp.zeros((B_BLK, GP), jnp.float32)
    acc = jnp.zeros((B_BLK, GP), jnp.float32)

    for k in range(min(_NSLOTS - 1, n_pairs)):
        for cp in pair_copies(k, k % _NSLOTS):
            cp.start()
    for k in range(n_pairs):
        kp = k + _NSLOTS - 1
        if kp < n_pairs:
            for cp in pair_copies(kp, kp % _NSLOTS):
                cp.start()
        for cp in pair_copies(k, k % _NSLOTS):
            cp.wait()
        p1 = jnp.dot(xbuf[k % _NSLOTS, 0], wia,
                     preferred_element_type=jnp.float32) + bias
        p2 = jnp.dot(xbuf[k % _NSLOTS, 1], wia,
                     preferred_element_type=jnp.float32) + bias
        gx_ref[n_pairs - 1 - k] = jnp.where(bwd_mask, p1, p2)
        gates = jnp.where(bwd_mask, p2, p1) + jnp.dot(
            h, whh, preferred_element_type=jnp.float32)
        h, c, acc = step(gates, h, c, acc)

    def tail(t, carry):
        h, c, acc = carry
        gates = gx_ref[t - n_pairs] + jnp.dot(
            h, whh, preferred_element_type=jnp.float32)
        return step(gates, h, c, acc)

    h, c, acc = jax.lax.fori_loop(n_pairs, T, tail, (h, c, acc),
                                  unroll=unroll)

    out_ref[...] = jnp.maximum(acc * inv_T, 0.0)


def kernel(x, wia, whh, b):
    B, T, D = x.shape
    GP = whh.shape[0]
    G = wia.shape[1]
    H = GP // 2

    col = jnp.arange(G) // GP
    scale = jnp.where(col == 2, 1.0, 0.5).astype(jnp.float32)
    wia_s = wia * scale
    whh_s = whh * scale
    b_s = b * scale

    Bp = max(8, -(-B // 8) * 8)
    B_BLK = min(Bp, 512)
    Bp = -(-Bp // B_BLK) * B_BLK
    if Bp != B:
        x = jnp.pad(x, ((0, Bp - B), (0, 0), (0, 0)))

    assert T % 2 == 0
    body = functools.partial(_bilstm_mean_relu_kernel, H=H, B_BLK=B_BLK,
                             unroll=4)

    out = pl.pallas_call(
        body,
        out_shape=jax.ShapeDtypeStruct((Bp, GP), jnp.float32),
        grid=(Bp // B_BLK,),
        in_specs=[
            pl.BlockSpec(memory_space=pltpu.MemorySpace.HBM),
            pl.BlockSpec(wia_s.shape, lambda i: (0, 0)),
            pl.BlockSpec(whh_s.shape, lambda i: (0, 0)),
            pl.BlockSpec(b_s.shape, lambda i: (0, 0)),
        ],
        out_specs=pl.BlockSpec((B_BLK, GP), lambda i: (i, 0)),
        scratch_shapes=[
            pltpu.VMEM((T // 2, B_BLK, G), jnp.float32),
            pltpu.VMEM((_NSLOTS, 2, B_BLK, D), jnp.float32),
            pltpu.SemaphoreType.DMA((_NSLOTS, 2)),
        ],
        compiler_params=pltpu.CompilerParams(
            dimension_semantics=("parallel",),
            vmem_limit_bytes=56 * 1024 * 1024,
        ),
    )(x, wia_s, whh_s, b_s)

    return out[:B, :GP]


# speedup vs baseline: 2.4215x; 1.1193x over previous
import functools

import jax
import jax.numpy as jnp
from jax.experimental import pallas as pl
from jax.experimental.pallas import tpu as pltpu

_NSLOTS = 4


def _bilstm_mean_relu_kernel(x_hbm, wia_ref, whh_ref, b_ref, out_ref,
                             gx_ref, xbuf, sem, *, H, B_BLK, unroll):
    _, T, _ = x_hbm.shape
    _, _, G = gx_ref.shape
    GP = G // 4
    inv_T = 1.0 / T
    base = pl.program_id(0) * B_BLK
    n_pairs = T // 2

    def pair_copies(k, slot):
        return (pltpu.make_async_copy(x_hbm.at[pl.ds(base, B_BLK), k, :],
                                      xbuf.at[slot, 0], sem.at[slot, 0]),
                pltpu.make_async_copy(x_hbm.at[pl.ds(base, B_BLK), T - 1 - k, :],
                                      xbuf.at[slot, 1], sem.at[slot, 1]))

    wia = wia_ref[...]
    bias = b_ref[...]

    lane = jax.lax.broadcasted_iota(jnp.int32, (1, G), 1) % GP
    bwd_mask = jnp.logical_and(lane >= H, lane < 2 * H)

    whh = whh_ref[...]

    def step(gates, h, c, acc):
        ti = jnp.tanh(gates[:, 0 * GP:1 * GP])
        tf = jnp.tanh(gates[:, 1 * GP:2 * GP])
        g = jnp.tanh(gates[:, 2 * GP:3 * GP])
        to = jnp.tanh(gates[:, 3 * GP:4 * GP])
        c = 0.5 * (tf * c + c + ti * g + g)
        tc = jnp.tanh(c)
        h = 0.5 * (to * tc + tc)
        return h, c, acc + h

    h = jnp.zeros((B_BLK, GP), jnp.float32)
    c = jnp.zeros((B_BLK, GP), jnp.float32)
    acc = jnp.zeros((B_BLK, GP), jnp.float32)

    for k in range(min(_NSLOTS - 1, n_pairs)):
        for cp in pair_copies(k, k % _NSLOTS):
            cp.start()
    for k in range(n_pairs):
        kp = k + _NSLOTS - 1
        if kp < n_pairs:
            for cp in pair_copies(kp, kp % _NSLOTS):
                cp.start()
        for cp in pair_copies(k, k % _NSLOTS):
            cp.wait()
        p1 = jnp.dot(xbuf[k % _NSLOTS, 0], wia,
                     preferred_element_type=jnp.float32) + bias
        p2 = jnp.dot(xbuf[k % _NSLOTS, 1], wia,
                     preferred_element_type=jnp.float32) + bias
        gx_ref[n_pairs - 1 - k] = jnp.where(bwd_mask, p1, p2)
        gates = jnp.where(bwd_mask, p2, p1) + jnp.dot(
            h, whh, preferred_element_type=jnp.float32)
        h, c, acc = step(gates, h, c, acc)

    def tail(t, carry):
        h, c, acc = carry
        gates = gx_ref[t - n_pairs] + jnp.dot(
            h, whh, preferred_element_type=jnp.float32)
        return step(gates, h, c, acc)

    h, c, acc = jax.lax.fori_loop(n_pairs, T, tail, (h, c, acc),
                                  unroll=unroll)

    out_ref[...] = jnp.maximum(acc * inv_T, 0.0)


def kernel(x, wia, whh, b):
    B, T, D = x.shape
    GP = whh.shape[0]
    G = wia.shape[1]
    H = GP // 2

    col = jnp.arange(G) // GP
    scale = jnp.where(col == 2, 1.0, 0.5).astype(jnp.float32)
    wia_s = wia * scale
    whh_s = whh * scale
    b_s = b * scale

    Bp = max(8, -(-B // 8) * 8)
    B_BLK = min(Bp, 1024)
    Bp = -(-Bp // B_BLK) * B_BLK
    if Bp != B:
        x = jnp.pad(x, ((0, Bp - B), (0, 0), (0, 0)))

    assert T % 2 == 0
    body = functools.partial(_bilstm_mean_relu_kernel, H=H, B_BLK=B_BLK,
                             unroll=4)

    out = pl.pallas_call(
        body,
        out_shape=jax.ShapeDtypeStruct((Bp, GP), jnp.float32),
        grid=(Bp // B_BLK,),
        in_specs=[
            pl.BlockSpec(memory_space=pltpu.MemorySpace.HBM),
            pl.BlockSpec(wia_s.shape, lambda i: (0, 0)),
            pl.BlockSpec(whh_s.shape, lambda i: (0, 0)),
            pl.BlockSpec(b_s.shape, lambda i: (0, 0)),
        ],
        out_specs=pl.BlockSpec((B_BLK, GP), lambda i: (i, 0)),
        scratch_shapes=[
            pltpu.VMEM((T // 2, B_BLK, G), jnp.float32),
            pltpu.VMEM((_NSLOTS, 2, B_BLK, D), jnp.float32),
            pltpu.SemaphoreType.DMA((_NSLOTS, 2)),
        ],
        compiler_params=pltpu.CompilerParams(
            dimension_semantics=("parallel",),
            vmem_limit_bytes=56 * 1024 * 1024,
        ),
    )(x, wia_s, whh_s, b_s)

    return out[:B, :GP]


# speedup vs baseline: 2.5307x; 1.0451x over previous
import functools

import jax
import jax.numpy as jnp
from jax.experimental import pallas as pl
from jax.experimental.pallas import tpu as pltpu

_NSLOTS = 4
_M = 256


def _bilstm_mean_relu_kernel(x_hbm, wia_ref, whh_ref, b_ref, out_ref,
                             gx_ref, xbuf, h_ref, c_ref, acc_ref, sem,
                             *, H, B_BLK):
    _, T, _ = x_hbm.shape
    _, _, G = gx_ref.shape
    GP = G // 4
    inv_T = 1.0 / T
    base = pl.program_id(0) * B_BLK
    n_pairs = T // 2
    chunks = [(s, min(_M, B_BLK - s)) for s in range(0, B_BLK, _M)]

    def pair_copies(k, slot):
        return (pltpu.make_async_copy(x_hbm.at[pl.ds(base, B_BLK), k, :],
                                      xbuf.at[slot, 0], sem.at[slot, 0]),
                pltpu.make_async_copy(x_hbm.at[pl.ds(base, B_BLK), T - 1 - k, :],
                                      xbuf.at[slot, 1], sem.at[slot, 1]))

    wia = wia_ref[...]
    bias = b_ref[...]
    whh = whh_ref[...]

    lane = jax.lax.broadcasted_iota(jnp.int32, (1, G), 1) % GP
    bwd_mask = jnp.logical_and(lane >= H, lane < 2 * H)

    h_ref[...] = jnp.zeros((B_BLK, GP), jnp.float32)
    c_ref[...] = jnp.zeros((B_BLK, GP), jnp.float32)
    acc_ref[...] = jnp.zeros((B_BLK, GP), jnp.float32)

    def step_chunk(gates, sl):
        ti = jnp.tanh(gates[:, 0 * GP:1 * GP])
        tf = jnp.tanh(gates[:, 1 * GP:2 * GP])
        g = jnp.tanh(gates[:, 2 * GP:3 * GP])
        to = jnp.tanh(gates[:, 3 * GP:4 * GP])
        c = c_ref[sl, :]
        c = 0.5 * (tf * c + c + ti * g + g)
        c_ref[sl, :] = c
        tc = jnp.tanh(c)
        hm = 0.5 * (to * tc + tc)
        h_ref[sl, :] = hm
        acc_ref[sl, :] += hm

    for k in range(min(_NSLOTS - 1, n_pairs)):
        for cp in pair_copies(k, k % _NSLOTS):
            cp.start()
    for k in range(n_pairs):
        kp = k + _NSLOTS - 1
        if kp < n_pairs:
            for cp in pair_copies(kp, kp % _NSLOTS):
                cp.start()
        for cp in pair_copies(k, k % _NSLOTS):
            cp.wait()
        for s, w in chunks:
            sl = pl.ds(s, w)
            p1 = jnp.dot(xbuf[k % _NSLOTS, 0, sl, :], wia,
                         preferred_element_type=jnp.float32) + bias
            p2 = jnp.dot(xbuf[k % _NSLOTS, 1, sl, :], wia,
                         preferred_element_type=jnp.float32) + bias
            gx_ref[n_pairs - 1 - k, sl, :] = jnp.where(bwd_mask, p1, p2)
            gates = jnp.where(bwd_mask, p2, p1) + jnp.dot(
                h_ref[sl, :], whh, preferred_element_type=jnp.float32)
            step_chunk(gates, sl)

    def tail(t, carry):
        for s, w in chunks:
            sl = pl.ds(s, w)
            gates = gx_ref[t - n_pairs, sl, :] + jnp.dot(
                h_ref[sl, :], whh, preferred_element_type=jnp.float32)
            step_chunk(gates, sl)
        return carry

    jax.lax.fori_loop(n_pairs, T, tail, 0, unroll=4)

    out_ref[...] = jnp.maximum(acc_ref[...] * inv_T, 0.0)


def kernel(x, wia, whh, b):
    B, T, D = x.shape
    GP = whh.shape[0]
    G = wia.shape[1]
    H = GP // 2

    col = jnp.arange(G) // GP
    scale = jnp.where(col == 2, 1.0, 0.5).astype(jnp.float32)
    wia_s = wia * scale
    whh_s = whh * scale
    b_s = b * scale

    Bp = max(8, -(-B // 8) * 8)
    B_BLK = min(Bp, 1024)
    Bp = -(-Bp // B_BLK) * B_BLK
    if Bp != B:
        x = jnp.pad(x, ((0, Bp - B), (0, 0), (0, 0)))

    assert T % 2 == 0
    body = functools.partial(_bilstm_mean_relu_kernel, H=H, B_BLK=B_BLK)

    out = pl.pallas_call(
        body,
        out_shape=jax.ShapeDtypeStruct((Bp, GP), jnp.float32),
        grid=(Bp // B_BLK,),
        in_specs=[
            pl.BlockSpec(memory_space=pltpu.MemorySpace.HBM),
            pl.BlockSpec(wia_s.shape, lambda i: (0, 0)),
            pl.BlockSpec(whh_s.shape, lambda i: (0, 0)),
            pl.BlockSpec(b_s.shape, lambda i: (0, 0)),
        ],
        out_specs=pl.BlockSpec((B_BLK, GP), lambda i: (i, 0)),
        scratch_shapes=[
            pltpu.VMEM((T // 2, B_BLK, G), jnp.float32),
            pltpu.VMEM((_NSLOTS, 2, B_BLK, D), jnp.float32),
            pltpu.VMEM((B_BLK, GP), jnp.float32),
            pltpu.VMEM((B_BLK, GP), jnp.float32),
            pltpu.VMEM((B_BLK, GP), jnp.float32),
            pltpu.SemaphoreType.DMA((_NSLOTS, 2)),
        ],
        compiler_params=pltpu.CompilerParams(
            dimension_semantics=("parallel",),
            vmem_limit_bytes=56 * 1024 * 1024,
        ),
    )(x, wia_s, whh_s, b_s)

    return out[:B, :GP]


# speedup vs baseline: 2.6267x; 1.0380x over previous
import functools

import jax
import jax.numpy as jnp
from jax.experimental import pallas as pl
from jax.experimental.pallas import tpu as pltpu

_NSLOTS = 4
_M = 128


def _bilstm_mean_relu_kernel(x_hbm, wia_ref, whh_ref, b_ref, out_ref,
                             gx_ref, xbuf, h_ref, c_ref, acc_ref, sem,
                             *, H, B_BLK):
    _, T, _ = x_hbm.shape
    _, _, G = gx_ref.shape
    GP = G // 4
    inv_T = 1.0 / T
    base = pl.program_id(0) * B_BLK
    n_pairs = T // 2
    chunks = [(s, min(_M, B_BLK - s)) for s in range(0, B_BLK, _M)]

    def pair_copies(k, slot):
        return (pltpu.make_async_copy(x_hbm.at[pl.ds(base, B_BLK), k, :],
                                      xbuf.at[slot, 0], sem.at[slot, 0]),
                pltpu.make_async_copy(x_hbm.at[pl.ds(base, B_BLK), T - 1 - k, :],
                                      xbuf.at[slot, 1], sem.at[slot, 1]))

    wia = wia_ref[...]
    bias = b_ref[...]
    whh = whh_ref[...]

    lane = jax.lax.broadcasted_iota(jnp.int32, (1, G), 1) % GP
    bwd_mask = jnp.logical_and(lane >= H, lane < 2 * H)

    h_ref[...] = jnp.zeros((B_BLK, GP), jnp.float32)
    c_ref[...] = jnp.zeros((B_BLK, GP), jnp.float32)
    acc_ref[...] = jnp.zeros((B_BLK, GP), jnp.float32)

    def step_chunk(gates, sl):
        ti = jnp.tanh(gates[:, 0 * GP:1 * GP])
        tf = jnp.tanh(gates[:, 1 * GP:2 * GP])
        g = jnp.tanh(gates[:, 2 * GP:3 * GP])
        to = jnp.tanh(gates[:, 3 * GP:4 * GP])
        c = c_ref[sl, :]
        c = 0.5 * (tf * c + c + ti * g + g)
        c_ref[sl, :] = c
        tc = jnp.tanh(c)
        hm = 0.5 * (to * tc + tc)
        h_ref[sl, :] = hm
        acc_ref[sl, :] += hm

    for k in range(min(_NSLOTS - 1, n_pairs)):
        for cp in pair_copies(k, k % _NSLOTS):
            cp.start()
    for k in range(n_pairs):
        kp = k + _NSLOTS - 1
        if kp < n_pairs:
            for cp in pair_copies(kp, kp % _NSLOTS):
                cp.start()
        for cp in pair_copies(k, k % _NSLOTS):
            cp.wait()
        for s, w in chunks:
            sl = pl.ds(s, w)
            p1 = jnp.dot(xbuf[k % _NSLOTS, 0, sl, :], wia,
                         preferred_element_type=jnp.float32) + bias
            p2 = jnp.dot(xbuf[k % _NSLOTS, 1, sl, :], wia,
                         preferred_element_type=jnp.float32) + bias
            gx_ref[n_pairs - 1 - k, sl, :] = jnp.where(bwd_mask, p1, p2)
            gates = jnp.where(bwd_mask, p2, p1) + jnp.dot(
                h_ref[sl, :], whh, preferred_element_type=jnp.float32)
            step_chunk(gates, sl)

    def tail(t, carry):
        for s, w in chunks:
            sl = pl.ds(s, w)
            gates = gx_ref[t - n_pairs, sl, :] + jnp.dot(
                h_ref[sl, :], whh, preferred_element_type=jnp.float32)
            step_chunk(gates, sl)
        return carry

    jax.lax.fori_loop(n_pairs, T, tail, 0, unroll=4)

    out_ref[...] = jnp.maximum(acc_ref[...] * inv_T, 0.0)


def kernel(x, wia, whh, b):
    B, T, D = x.shape
    GP = whh.shape[0]
    G = wia.shape[1]
    H = GP // 2

    col = jnp.arange(G) // GP
    scale = jnp.where(col == 2, 1.0, 0.5).astype(jnp.float32)
    wia_s = wia * scale
    whh_s = whh * scale
    b_s = b * scale

    Bp = max(8, -(-B // 8) * 8)
    B_BLK = min(Bp, 1024)
    Bp = -(-Bp // B_BLK) * B_BLK
    if Bp != B:
        x = jnp.pad(x, ((0, Bp - B), (0, 0), (0, 0)))

    assert T % 2 == 0
    body = functools.partial(_bilstm_mean_relu_kernel, H=H, B_BLK=B_BLK)

    out = pl.pallas_call(
        body,
        out_shape=jax.ShapeDtypeStruct((Bp, GP), jnp.float32),
        grid=(Bp // B_BLK,),
        in_specs=[
            pl.BlockSpec(memory_space=pltpu.MemorySpace.HBM),
            pl.BlockSpec(wia_s.shape, lambda i: (0, 0)),
            pl.BlockSpec(whh_s.shape, lambda i: (0, 0)),
            pl.BlockSpec(b_s.shape, lambda i: (0, 0)),
        ],
        out_specs=pl.BlockSpec((B_BLK, GP), lambda i: (i, 0)),
        scratch_shapes=[
            pltpu.VMEM((T // 2, B_BLK, G), jnp.float32),
            pltpu.VMEM((_NSLOTS, 2, B_BLK, D), jnp.float32),
            pltpu.VMEM((B_BLK, GP), jnp.float32),
            pltpu.VMEM((B_BLK, GP), jnp.float32),
            pltpu.VMEM((B_BLK, GP), jnp.float32),
            pltpu.SemaphoreType.DMA((_NSLOTS, 2)),
        ],
        compiler_params=pltpu.CompilerParams(
            dimension_semantics=("parallel",),
            vmem_limit_bytes=56 * 1024 * 1024,
        ),
    )(x, wia_s, whh_s, b_s)

    return out[:B, :GP]
